# Initial kernel scaffold; baseline (speedup 1.0000x reference)
import jax
import jax.numpy as jnp
from jax.experimental import pallas as pl


def kernel(A, tau_init, G_latent_init):
    raise NotImplementedError("write your pallas kernel here")


# speedup vs baseline: 5.1194x; 5.1194x over previous
import jax
import jax.numpy as jnp
from jax.experimental import pallas as pl
from jax.experimental.pallas import tpu as pltpu

_N = 256
_NM = 248
_PAD = _N - _NM
_ITERS = 3
_LR = 0.1
_B1, _B2, _EPS = 0.9, 0.999, 1e-8
_UB = 8
_NBLK = _NM // _UB

_INV_2NM = 1.0 / (2.0 * _NM)
_INV_MEAN = 1.0 / (_NM * _N)


def _body(aw_ref, hlat0_ref, tau0_ref, out_ref,
          hlat_ref, h_ref, gh_ref, mg_ref, vg_ref,
          tau_ref, mt_ref, vt_ref):
    aw = aw_ref[...]

    lane = jax.lax.broadcasted_iota(jnp.int32, (_NM, _N), 1)
    colmask = lane >= _PAD

    hlat_ref[...] = hlat0_ref[...]
    tau_ref[...] = tau0_ref[...]
    mg_ref[...] = jnp.zeros((_NM, _N), jnp.float32)
    vg_ref[...] = jnp.zeros((_NM, _N), jnp.float32)
    mt_ref[...] = jnp.zeros((1, _N), jnp.float32)
    vt_ref[...] = jnp.zeros((1, _N), jnp.float32)

    def lane_prod(x):
        w = x.shape[-1]
        while w > 1:
            w //= 2
            x = x[..., :w] * x[..., w:]
        return x

    def block_products(u0):
        h_blk = h_ref[pl.ds(u0, _UB), :]
        f = 1.0 - 2.0 * aw[None, :, :] * h_blk[:, None, :]
        fz = f == 0.0
        f_safe = jnp.where(fz, 1.0, f)
        nzprod = lane_prod(f_safe)
        zcnt = jnp.sum(jnp.where(fz, 1.0, 0.0), axis=2, keepdims=True)
        products = jnp.where(zcnt == 0.0, nzprod, 0.0)
        w_iota = jax.lax.broadcasted_iota(jnp.int32, (_UB, _NM, 1), 1)
        u_iota = jax.lax.broadcasted_iota(jnp.int32, (_UB, _NM, 1), 0)
        tcol = jnp.where(w_iota == u_iota + u0, -1.0, 1.0)
        return f_safe, fz, nzprod, zcnt, products, tcol

    def grad_step(t):
        hlat = hlat_ref[...]
        h = jnp.where(colmask, jax.nn.sigmoid(hlat), 0.0)
        h_ref[...] = h

        def blk(i, carry):
            u0 = i * _UB
            f_safe, fz, nzprod, zcnt, products, tcol = block_products(u0)
            gprod = (products - tcol) * _INV_2NM
            prodrest = nzprod * jnp.where(
                zcnt == 0.0, 1.0 / f_safe,
                jnp.where(jnp.logical_and(zcnt == 1.0, fz), 1.0, 0.0))
            contrib = (gprod * prodrest) * aw[None, :, :]
            gh_rows = -2.0 * jnp.sum(contrib, axis=1)
            gh_ref[pl.ds(u0, _UB), :] = gh_rows
            return carry

        jax.lax.fori_loop(0, _NBLK, blk, 0, unroll=False)

        tau = tau_ref[...]
        tau_col = jnp.transpose(tau)[: _NM, :]
        d = tau_col - tau + 0.1
        r = jnp.maximum(d, 0.0)
        h = h_ref[...]
        gh = gh_ref[...] + (r * r) * _INV_MEAN
        ghlat = gh * (h * (1.0 - h))

        w_mat = h * (2.0 * r)
        rowsum = jnp.sum(w_mat, axis=1, keepdims=True)
        rowsum_full = jnp.concatenate(
            [rowsum, jnp.zeros((_PAD, 1), jnp.float32)], axis=0)
        gtau = (jnp.transpose(rowsum_full)
                - jnp.sum(w_mat, axis=0, keepdims=True)) * _INV_MEAN

        c1 = 1.0 - _B1 ** t
        c2 = 1.0 - _B2 ** t
        mg = _B1 * mg_ref[...] + (1.0 - _B1) * ghlat
        vg = _B2 * vg_ref[...] + (1.0 - _B2) * (ghlat * ghlat)
        mg_ref[...] = mg
        vg_ref[...] = vg
        hlat_ref[...] = hlat - _LR * (mg / c1) / (jnp.sqrt(vg / c2) + _EPS)

        mt = _B1 * mt_ref[...] + (1.0 - _B1) * gtau
        vt = _B2 * vt_ref[...] + (1.0 - _B2) * (gtau * gtau)
        mt_ref[...] = mt
        vt_ref[...] = vt
        tau_ref[...] = tau - _LR * (mt / c1) / (jnp.sqrt(vt / c2) + _EPS)

    for t in range(1, _ITERS + 1):
        grad_step(t)

    h = jnp.where(colmask, jax.nn.sigmoid(hlat_ref[...]), 0.0)
    h_ref[...] = h

    def loss_blk(i, acc):
        u0 = i * _UB
        _, _, _, _, products, tcol = block_products(u0)
        se = (products - tcol) ** 2
        return acc + jnp.sum(se)

    odd_sum = jax.lax.fori_loop(0, _NBLK, loss_blk, jnp.float32(0.0),
                                unroll=False)
    loss_odd = odd_sum / (4.0 * _NM)

    tau = tau_ref[...]
    tau_col = jnp.transpose(tau)[: _NM, :]
    r = jnp.maximum(tau_col - tau + 0.1, 0.0)
    loss_order = jnp.sum(h * (r * r)) * _INV_MEAN
    out_ref[0, 0] = loss_odd + loss_order


def kernel(A, tau_init, G_latent_init):
    aw = A[: _NM, :]
    hlat0 = jnp.pad(G_latent_init, ((0, 0), (_PAD, 0)))
    tau0 = tau_init.reshape(1, _N)

    out = pl.pallas_call(
        _body,
        out_shape=jax.ShapeDtypeStruct((1, 1), jnp.float32),
        out_specs=pl.BlockSpec(memory_space=pltpu.SMEM),
        scratch_shapes=[
            pltpu.VMEM((_NM, _N), jnp.float32),
            pltpu.VMEM((_NM, _N), jnp.float32),
            pltpu.VMEM((_NM, _N), jnp.float32),
            pltpu.VMEM((_NM, _N), jnp.float32),
            pltpu.VMEM((_NM, _N), jnp.float32),
            pltpu.VMEM((1, _N), jnp.float32),
            pltpu.VMEM((1, _N), jnp.float32),
            pltpu.VMEM((1, _N), jnp.float32),
        ],
    )(aw, hlat0, tau0)
    return out[0, 0]


# speedup vs baseline: 6.5100x; 1.2716x over previous
import functools

import jax
import jax.numpy as jnp
import numpy as np
from jax.experimental import pallas as pl
from jax.experimental.pallas import tpu as pltpu

_N = 256
_NM = 248
_PAD = _N - _NM
_ITERS = 3
_LR = 0.1
_B1, _B2, _EPS = 0.9, 0.999, 1e-8
_UB = 8
_NBLK = _NM // _UB

_INV_2NM = 1.0 / (2.0 * _NM)
_INV_MEAN = 1.0 / (_NM * _N)


def _lane_prod(x):
    w = x.shape[-1]
    while w > 1:
        w //= 2
        x = x[..., :w] * x[..., w:]
    return x


def _masked_sigmoid(hlat):
    lane = jax.lax.broadcasted_iota(jnp.int32, hlat.shape, hlat.ndim - 1)
    return jnp.where(lane >= _PAD, jax.nn.sigmoid(hlat), 0.0)


def _block_targets(u0):
    w_io = jax.lax.broadcasted_iota(jnp.int32, (_UB, _NM, 1), 1)
    u_io = jax.lax.broadcasted_iota(jnp.int32, (_UB, _NM, 1), 0)
    return jnp.where(w_io == u_io + u0, -1.0, 1.0)


def _grad_odd_body(aw2_ref, hlat_ref, gh_ref):
    u0 = pl.program_id(0) * _UB
    aw2 = aw2_ref[...]
    h = _masked_sigmoid(hlat_ref[...])
    f = 1.0 - aw2[None, :, :] * h[:, None, :]
    fz = f == 0.0
    zc = jnp.sum(jnp.where(fz, 1.0, 0.0), axis=2, keepdims=True)
    praw = _lane_prod(f)
    tcol = _block_targets(u0)
    gprod = (praw - tcol) * _INV_2NM

    def fast(_):
        contrib = ((gprod * praw) / f) * aw2[None, :, :]
        return -jnp.sum(contrib, axis=1)

    def slow(_):
        f_safe = jnp.where(fz, 1.0, f)
        nzprod = _lane_prod(f_safe)
        q = jnp.where(fz, jnp.where(zc == 1.0, 1.0, 0.0),
                      jnp.where(zc == 0.0, 1.0 / f_safe, 0.0))
        contrib = ((gprod * nzprod) * q) * aw2[None, :, :]
        return -jnp.sum(contrib, axis=1)

    gh_ref[...] = jax.lax.cond(jnp.max(zc) > 0.0, slow, fast, 0)


def _update_body(cc_ref, hlat_ref, tau_ref, mg_ref, vg_ref, mt_ref, vt_ref,
                 gh_ref, hlat_o, tau_o, mg_o, vg_o, mt_o, vt_o):
    c1 = cc_ref[0]
    c2 = cc_ref[1]
    hlat = hlat_ref[...]
    h = _masked_sigmoid(hlat)
    tau = tau_ref[...]
    tau_col = jnp.transpose(tau)[:_NM, :]
    r = jnp.maximum(tau_col - tau + 0.1, 0.0)
    gh = gh_ref[...] + (r * r) * _INV_MEAN
    ghlat = gh * (h * (1.0 - h))

    w_mat = h * (2.0 * r)
    rowsum = jnp.sum(w_mat, axis=1, keepdims=True)
    rowsum_full = jnp.concatenate(
        [rowsum, jnp.zeros((_PAD, 1), jnp.float32)], axis=0)
    gtau = (jnp.transpose(rowsum_full)
            - jnp.sum(w_mat, axis=0, keepdims=True)) * _INV_MEAN

    mg = _B1 * mg_ref[...] + (1.0 - _B1) * ghlat
    vg = _B2 * vg_ref[...] + (1.0 - _B2) * (ghlat * ghlat)
    mg_o[...] = mg
    vg_o[...] = vg
    hlat_o[...] = hlat - _LR * (mg / c1) / (jnp.sqrt(vg / c2) + _EPS)

    mt = _B1 * mt_ref[...] + (1.0 - _B1) * gtau
    vt = _B2 * vt_ref[...] + (1.0 - _B2) * (gtau * gtau)
    mt_o[...] = mt
    vt_o[...] = vt
    tau_o[...] = tau - _LR * (mt / c1) / (jnp.sqrt(vt / c2) + _EPS)


def _loss_body(aw2_ref, hlat_ref, tau_ref, taucol_ref, out_ref):
    u0 = pl.program_id(0) * _UB
    aw2 = aw2_ref[...]
    h = _masked_sigmoid(hlat_ref[...])
    f = 1.0 - aw2[None, :, :] * h[:, None, :]
    praw = _lane_prod(f)
    se = (praw - _block_targets(u0)) ** 2
    odd = jnp.sum(se)
    r = jnp.maximum(taucol_ref[...] - tau_ref[...] + 0.1, 0.0)
    order = jnp.sum(h * (r * r))
    out_ref[0, 0, 0] = odd / (4.0 * _NM) + order * _INV_MEAN


def _combine_body(p_ref, out_ref):
    out_ref[0, 0] = jnp.sum(p_ref[...])


_PARALLEL = pltpu.CompilerParams(dimension_semantics=("parallel",))


@functools.lru_cache(maxsize=None)
def _build_calls():
    grad_call = pl.pallas_call(
        _grad_odd_body,
        grid=(_NBLK,),
        in_specs=[
        pl.BlockSpec((_NM, _N), lambda i: (0, 0)),
        pl.BlockSpec((_UB, _N), lambda i: (i, 0)),
        ],
        out_specs=pl.BlockSpec((_UB, _N), lambda i: (i, 0)),
        out_shape=jax.ShapeDtypeStruct((_NM, _N), jnp.float32),
        compiler_params=_PARALLEL,
    )

    update_call = pl.pallas_call(
        _update_body,
        in_specs=[
        pl.BlockSpec(memory_space=pltpu.SMEM),
        pl.BlockSpec((_NM, _N), lambda: (0, 0)),
        pl.BlockSpec((1, _N), lambda: (0, 0)),
        pl.BlockSpec((_NM, _N), lambda: (0, 0)),
        pl.BlockSpec((_NM, _N), lambda: (0, 0)),
        pl.BlockSpec((1, _N), lambda: (0, 0)),
        pl.BlockSpec((1, _N), lambda: (0, 0)),
        pl.BlockSpec((_NM, _N), lambda: (0, 0)),
        ],
        out_specs=[
        pl.BlockSpec((_NM, _N), lambda: (0, 0)),
        pl.BlockSpec((1, _N), lambda: (0, 0)),
        pl.BlockSpec((_NM, _N), lambda: (0, 0)),
        pl.BlockSpec((_NM, _N), lambda: (0, 0)),
        pl.BlockSpec((1, _N), lambda: (0, 0)),
        pl.BlockSpec((1, _N), lambda: (0, 0)),
        ],
        out_shape=[
        jax.ShapeDtypeStruct((_NM, _N), jnp.float32),
        jax.ShapeDtypeStruct((1, _N), jnp.float32),
        jax.ShapeDtypeStruct((_NM, _N), jnp.float32),
        jax.ShapeDtypeStruct((_NM, _N), jnp.float32),
        jax.ShapeDtypeStruct((1, _N), jnp.float32),
        jax.ShapeDtypeStruct((1, _N), jnp.float32),
        ],
        input_output_aliases={1: 0, 2: 1, 3: 2, 4: 3, 5: 4, 6: 5},
    )

    loss_call = pl.pallas_call(
        _loss_body,
        grid=(_NBLK,),
        in_specs=[
        pl.BlockSpec((_NM, _N), lambda i: (0, 0)),
        pl.BlockSpec((_UB, _N), lambda i: (i, 0)),
        pl.BlockSpec((1, _N), lambda i: (0, 0)),
        pl.BlockSpec((_UB, 1), lambda i: (i, 0)),
        ],
        out_specs=pl.BlockSpec((1, 1, 1), lambda i: (i, 0, 0),
                               memory_space=pltpu.SMEM),
        out_shape=jax.ShapeDtypeStruct((_NBLK, 1, 1), jnp.float32),
        compiler_params=_PARALLEL,
    )

    combine_call = pl.pallas_call(
        _combine_body,
        in_specs=[pl.BlockSpec((_NBLK, 1), lambda: (0, 0))],
        out_specs=pl.BlockSpec(memory_space=pltpu.SMEM),
        out_shape=jax.ShapeDtypeStruct((1, 1), jnp.float32),
    )
    return grad_call, update_call, loss_call, combine_call


def kernel(A, tau_init, G_latent_init):
    aw2 = 2.0 * A[: _NM, :]
    hlat = jnp.pad(G_latent_init, ((0, 0), (_PAD, 0)))
    tau = tau_init.reshape(1, _N)
    z_g = jnp.zeros((_NM, _N), jnp.float32)
    z_t = jnp.zeros((1, _N), jnp.float32)
    mg, vg, mt, vt = z_g, z_g, z_t, z_t

    grad_call, update_call, loss_call, combine_call = _build_calls()
    for t in range(1, _ITERS + 1):
        gh = grad_call(aw2, hlat)
        cc = jnp.asarray(
            np.array([1.0 - _B1 ** t, 1.0 - _B2 ** t], dtype=np.float32))
        hlat, tau, mg, vg, mt, vt = update_call(
            cc, hlat, tau, mg, vg, mt, vt, gh)

    partials = loss_call(aw2, hlat, tau, tau.reshape(_N, 1))
    return combine_call(partials.reshape(_NBLK, 1))[0, 0]


# speedup vs baseline: 6.5130x; 1.0005x over previous
import functools

import jax
import jax.numpy as jnp
import numpy as np
from jax.experimental import pallas as pl
from jax.experimental.pallas import tpu as pltpu

_N = 256
_NM = 248
_PAD = _N - _NM
_ITERS = 3
_LR = 0.1
_B1, _B2, _EPS = 0.9, 0.999, 1e-8
_UB = 8
_NBLK = _NM // _UB

_INV_2NM = 1.0 / (2.0 * _NM)
_INV_MEAN = 1.0 / (_NM * _N)


def _lane_prod(x):
    w = x.shape[-1]
    while w > 1:
        w //= 2
        x = x[..., :w] * x[..., w:]
    return x


def _masked_sigmoid(hlat):
    lane = jax.lax.broadcasted_iota(jnp.int32, hlat.shape, hlat.ndim - 1)
    return jnp.where(lane >= _PAD, jax.nn.sigmoid(hlat), 0.0)


def _block_targets(u0):
    w_io = jax.lax.broadcasted_iota(jnp.int32, (_UB, _NM, 1), 1)
    u_io = jax.lax.broadcasted_iota(jnp.int32, (_UB, _NM, 1), 0)
    return jnp.where(w_io == u_io + u0, -1.0, 1.0)


def _grad_odd_body(aw2_ref, hlat_ref, gh_ref):
    u0 = pl.program_id(0) * _UB
    aw2 = aw2_ref[...]
    h = _masked_sigmoid(hlat_ref[...])
    f = 1.0 - aw2[None, :, :] * h[:, None, :]
    fz = f == 0.0
    zc = jnp.sum(jnp.where(fz, 1.0, 0.0), axis=2, keepdims=True)
    praw = _lane_prod(f)
    tcol = _block_targets(u0)
    gprod = (praw - tcol) * _INV_2NM

    def fast(_):
        contrib = ((gprod * praw) / f) * aw2[None, :, :]
        return -jnp.sum(contrib, axis=1)

    def slow(_):
        f_safe = jnp.where(fz, 1.0, f)
        nzprod = _lane_prod(f_safe)
        q = jnp.where(fz, jnp.where(zc == 1.0, 1.0, 0.0),
                      jnp.where(zc == 0.0, 1.0 / f_safe, 0.0))
        contrib = ((gprod * nzprod) * q) * aw2[None, :, :]
        return -jnp.sum(contrib, axis=1)

    gh_ref[...] = jax.lax.cond(jnp.max(zc) > 0.0, slow, fast, 0)


def _update_body(cc_ref, hlat_ref, tau_ref, mg_ref, vg_ref, mt_ref, vt_ref,
                 gh_ref, hlat_o, tau_o, mg_o, vg_o, mt_o, vt_o):
    c1 = cc_ref[0]
    c2 = cc_ref[1]
    hlat = hlat_ref[...]
    h = _masked_sigmoid(hlat)
    tau = tau_ref[...]
    tau_col = jnp.transpose(tau)[:_NM, :]
    r = jnp.maximum(tau_col - tau + 0.1, 0.0)
    gh = gh_ref[...] + (r * r) * _INV_MEAN
    ghlat = gh * (h * (1.0 - h))

    w_mat = h * (2.0 * r)
    rowsum = jnp.sum(w_mat, axis=1, keepdims=True)
    rowsum_full = jnp.concatenate(
        [rowsum, jnp.zeros((_PAD, 1), jnp.float32)], axis=0)
    gtau = (jnp.transpose(rowsum_full)
            - jnp.sum(w_mat, axis=0, keepdims=True)) * _INV_MEAN

    mg = _B1 * mg_ref[...] + (1.0 - _B1) * ghlat
    vg = _B2 * vg_ref[...] + (1.0 - _B2) * (ghlat * ghlat)
    mg_o[...] = mg
    vg_o[...] = vg
    hlat_o[...] = hlat - _LR * (mg / c1) / (jnp.sqrt(vg / c2) + _EPS)

    mt = _B1 * mt_ref[...] + (1.0 - _B1) * gtau
    vt = _B2 * vt_ref[...] + (1.0 - _B2) * (gtau * gtau)
    mt_o[...] = mt
    vt_o[...] = vt
    tau_o[...] = tau - _LR * (mt / c1) / (jnp.sqrt(vt / c2) + _EPS)


def _loss_body(aw2_ref, hlat_ref, tau_ref, taucol_ref, out_ref):
    u0 = pl.program_id(0) * _UB
    aw2 = aw2_ref[...]
    h = _masked_sigmoid(hlat_ref[...])
    f = 1.0 - aw2[None, :, :] * h[:, None, :]
    praw = _lane_prod(f)
    se = (praw - _block_targets(u0)) ** 2
    odd = jnp.sum(se)
    r = jnp.maximum(taucol_ref[...] - tau_ref[...] + 0.1, 0.0)
    order = jnp.sum(h * (r * r))
    out_ref[0, 0, 0] = odd / (4.0 * _NM) + order * _INV_MEAN


def _combine_body(p_ref, out_ref):
    out_ref[0, 0] = jnp.sum(p_ref[...])


_PARALLEL = pltpu.CompilerParams(dimension_semantics=("arbitrary",))


@functools.lru_cache(maxsize=None)
def _build_calls():
    grad_call = pl.pallas_call(
        _grad_odd_body,
        grid=(_NBLK,),
        in_specs=[
        pl.BlockSpec((_NM, _N), lambda i: (0, 0)),
        pl.BlockSpec((_UB, _N), lambda i: (i, 0)),
        ],
        out_specs=pl.BlockSpec((_UB, _N), lambda i: (i, 0)),
        out_shape=jax.ShapeDtypeStruct((_NM, _N), jnp.float32),
        compiler_params=_PARALLEL,
    )

    update_call = pl.pallas_call(
        _update_body,
        in_specs=[
        pl.BlockSpec(memory_space=pltpu.SMEM),
        pl.BlockSpec((_NM, _N), lambda: (0, 0)),
        pl.BlockSpec((1, _N), lambda: (0, 0)),
        pl.BlockSpec((_NM, _N), lambda: (0, 0)),
        pl.BlockSpec((_NM, _N), lambda: (0, 0)),
        pl.BlockSpec((1, _N), lambda: (0, 0)),
        pl.BlockSpec((1, _N), lambda: (0, 0)),
        pl.BlockSpec((_NM, _N), lambda: (0, 0)),
        ],
        out_specs=[
        pl.BlockSpec((_NM, _N), lambda: (0, 0)),
        pl.BlockSpec((1, _N), lambda: (0, 0)),
        pl.BlockSpec((_NM, _N), lambda: (0, 0)),
        pl.BlockSpec((_NM, _N), lambda: (0, 0)),
        pl.BlockSpec((1, _N), lambda: (0, 0)),
        pl.BlockSpec((1, _N), lambda: (0, 0)),
        ],
        out_shape=[
        jax.ShapeDtypeStruct((_NM, _N), jnp.float32),
        jax.ShapeDtypeStruct((1, _N), jnp.float32),
        jax.ShapeDtypeStruct((_NM, _N), jnp.float32),
        jax.ShapeDtypeStruct((_NM, _N), jnp.float32),
        jax.ShapeDtypeStruct((1, _N), jnp.float32),
        jax.ShapeDtypeStruct((1, _N), jnp.float32),
        ],
        input_output_aliases={1: 0, 2: 1, 3: 2, 4: 3, 5: 4, 6: 5},
    )

    loss_call = pl.pallas_call(
        _loss_body,
        grid=(_NBLK,),
        in_specs=[
        pl.BlockSpec((_NM, _N), lambda i: (0, 0)),
        pl.BlockSpec((_UB, _N), lambda i: (i, 0)),
        pl.BlockSpec((1, _N), lambda i: (0, 0)),
        pl.BlockSpec((_UB, 1), lambda i: (i, 0)),
        ],
        out_specs=pl.BlockSpec((1, 1, 1), lambda i: (i, 0, 0),
                               memory_space=pltpu.SMEM),
        out_shape=jax.ShapeDtypeStruct((_NBLK, 1, 1), jnp.float32),
        compiler_params=_PARALLEL,
    )

    combine_call = pl.pallas_call(
        _combine_body,
        in_specs=[pl.BlockSpec((_NBLK, 1), lambda: (0, 0))],
        out_specs=pl.BlockSpec(memory_space=pltpu.SMEM),
        out_shape=jax.ShapeDtypeStruct((1, 1), jnp.float32),
    )
    return grad_call, update_call, loss_call, combine_call


def kernel(A, tau_init, G_latent_init):
    aw2 = 2.0 * A[: _NM, :]
    hlat = jnp.pad(G_latent_init, ((0, 0), (_PAD, 0)))
    tau = tau_init.reshape(1, _N)
    z_g = jnp.zeros((_NM, _N), jnp.float32)
    z_t = jnp.zeros((1, _N), jnp.float32)
    mg, vg, mt, vt = z_g, z_g, z_t, z_t

    grad_call, update_call, loss_call, combine_call = _build_calls()
    for t in range(1, _ITERS + 1):
        gh = grad_call(aw2, hlat)
        cc = jnp.asarray(
            np.array([1.0 - _B1 ** t, 1.0 - _B2 ** t], dtype=np.float32))
        hlat, tau, mg, vg, mt, vt = update_call(
            cc, hlat, tau, mg, vg, mt, vt, gh)

    partials = loss_call(aw2, hlat, tau, tau.reshape(_N, 1))
    return combine_call(partials.reshape(_NBLK, 1))[0, 0]
